# baseline (speedup 1.0000x reference)
import functools

import jax
import jax.numpy as jnp
from jax import lax
from jax.experimental import pallas as pl
from jax.experimental.pallas import tpu as pltpu
from jax.experimental.pallas import tpu_sc as plsc

_LANES = 16
_SCATTER_BATCH = 128


@functools.lru_cache(maxsize=None)
def _make_sc_segsum(n_edges: int, n_nodes: int, d_edge: int, with_init: bool):
    info = plsc.get_sparse_core_info()
    nc, ns = info.num_cores, info.num_subcores
    nw = nc * ns
    rows = n_edges // _SCATTER_BATCH
    assert rows * _SCATTER_BATCH == n_edges
    rpt = rows // nw
    extra = rows - rpt * nw
    stripe = (n_nodes // ns) // 8 * 8
    tail = n_nodes - ns * stripe
    assert tail % 8 == 0
    hs = 13
    chunks = []
    r = 0
    while r < rpt:
        n = min(hs, rpt - r)
        chunks.append((r, n))
        r += n

    mesh = plsc.VectorSubcoreMesh(core_axis_name="c", subcore_axis_name="s")
    f32 = jnp.float32

    def body(pair_hbm, idx_hbm, acc_in, cnt_in, acc_out, cnt_out,
             acc_sh, cnt_sh, idx_v, pair_v, ones_v, zero_v, eidx_v, epair_v,
             lsem, ssem):
        c = lax.axis_index("c")
        s = lax.axis_index("s")
        wid = c * ns + s

        idx_h = pltpu.async_copy(
            idx_hbm.at[pl.ds(wid * rpt * _SCATTER_BATCH, rpt * _SCATTER_BATCH)],
            idx_v, lsem)

        def start_load(ci):
            r0, n = chunks[ci]
            return pltpu.async_copy(
                pair_hbm.at[pl.ds((wid * rpt + r0) * _SCATTER_BATCH,
                                  n * _SCATTER_BATCH)],
                pair_v.at[ci % 2, pl.ds(0, n * _SCATTER_BATCH)], lsem)

        loads = [start_load(0), None]

        def fill(i, _):
            zero_v[i] = jnp.zeros((_LANES,), f32)
            return 0

        def fill1(i, _):
            ones_v[i] = jnp.ones((_LANES,), f32)
            return 0
        lax.fori_loop(0, _SCATTER_BATCH, fill1, 0)

        if with_init:
            pltpu.sync_copy(acc_in.at[c, pl.ds(s * stripe, stripe)],
                            acc_sh.at[pl.ds(s * stripe, stripe)])
            pltpu.sync_copy(cnt_in.at[c, pl.ds(s * stripe, stripe)],
                            cnt_sh.at[pl.ds(s * stripe, stripe)])

            @pl.when(s == 0)
            def _():
                pltpu.sync_copy(acc_in.at[c, pl.ds(ns * stripe, tail)],
                                acc_sh.at[pl.ds(ns * stripe, tail)])
                pltpu.sync_copy(cnt_in.at[c, pl.ds(ns * stripe, tail)],
                                cnt_sh.at[pl.ds(ns * stripe, tail)])
        else:
            lax.fori_loop(0, stripe, fill, 0)
            pltpu.sync_copy(zero_v, acc_sh.at[pl.ds(s * stripe, stripe)])
            pltpu.sync_copy(zero_v, cnt_sh.at[pl.ds(s * stripe, stripe)])

            @pl.when(s == 0)
            def _():
                pltpu.sync_copy(zero_v.at[pl.ds(0, tail)],
                                acc_sh.at[pl.ds(ns * stripe, tail)])
                pltpu.sync_copy(zero_v.at[pl.ds(0, tail)],
                                cnt_sh.at[pl.ds(ns * stripe, tail)])

        plsc.subcore_barrier()
        idx_h.wait()

        for ci, (r0, n) in enumerate(chunks):
            loads[ci % 2].wait()
            if ci + 1 < len(chunks):
                loads[(ci + 1) % 2] = start_load(ci + 1)
            handles = []
            for j in range(n):
                idx_row = idx_v.at[pl.ds((r0 + j) * _SCATTER_BATCH,
                                         _SCATTER_BATCH)]
                handles.append(pltpu.async_copy(
                    pair_v.at[ci % 2, pl.ds(j * _SCATTER_BATCH, _SCATTER_BATCH)],
                    acc_sh.at[idx_row], ssem, add=True))
                handles.append(pltpu.async_copy(
                    ones_v, cnt_sh.at[idx_row], ssem, add=True))
            for h in handles:
                h.wait()

        if extra:
            @pl.when(wid < extra)
            def _():
                e_base = (nw * rpt + wid) * _SCATTER_BATCH
                pltpu.sync_copy(idx_hbm.at[pl.ds(e_base, _SCATTER_BATCH)], eidx_v)
                pltpu.sync_copy(pair_hbm.at[pl.ds(e_base, _SCATTER_BATCH)],
                                epair_v)
                pltpu.sync_copy(epair_v, acc_sh.at[eidx_v], add=True)
                pltpu.sync_copy(ones_v, cnt_sh.at[eidx_v], add=True)

        plsc.subcore_barrier()

        pltpu.sync_copy(acc_sh.at[pl.ds(s * stripe, stripe)],
                        acc_out.at[c, pl.ds(s * stripe, stripe)])
        pltpu.sync_copy(cnt_sh.at[pl.ds(s * stripe, stripe)],
                        cnt_out.at[c, pl.ds(s * stripe, stripe)])

        @pl.when(s == 0)
        def _():
            pltpu.sync_copy(acc_sh.at[pl.ds(ns * stripe, tail)],
                            acc_out.at[c, pl.ds(ns * stripe, tail)])
            pltpu.sync_copy(cnt_sh.at[pl.ds(ns * stripe, tail)],
                            cnt_out.at[c, pl.ds(ns * stripe, tail)])

    scratch = [
        pltpu.VMEM_SHARED((n_nodes, d_edge), f32),
        pltpu.VMEM_SHARED((n_nodes, d_edge), f32),
        pltpu.VMEM((rpt * _SCATTER_BATCH,), jnp.int32),
        pltpu.VMEM((2, hs * _SCATTER_BATCH, d_edge), f32),
        pltpu.VMEM((_SCATTER_BATCH, d_edge), f32),
        pltpu.VMEM((stripe, d_edge), f32),
        pltpu.VMEM((_SCATTER_BATCH,), jnp.int32),
        pltpu.VMEM((_SCATTER_BATCH, d_edge), f32),
        pltpu.SemaphoreType.DMA,
        pltpu.SemaphoreType.DMA,
    ]
    out_type = (
        jax.ShapeDtypeStruct((nc, n_nodes, d_edge), f32),
        jax.ShapeDtypeStruct((nc, n_nodes, d_edge), f32),
    )

    if with_init:
        @functools.partial(pl.kernel, mesh=mesh,
                           compiler_params=pltpu.CompilerParams(
                               use_tc_tiling_on_sc=False),
                           out_type=out_type, scratch_types=scratch)
        def sc_segsum(pair_hbm, idx_hbm, acc_in, cnt_in, acc_out, cnt_out, *sc):
            body(pair_hbm, idx_hbm, acc_in, cnt_in, acc_out, cnt_out, *sc)
    else:
        @functools.partial(pl.kernel, mesh=mesh,
                           compiler_params=pltpu.CompilerParams(
                               use_tc_tiling_on_sc=False),
                           out_type=out_type, scratch_types=scratch)
        def sc_segsum(pair_hbm, idx_hbm, acc_out, cnt_out, *sc):
            body(pair_hbm, idx_hbm, None, None, acc_out, cnt_out, *sc)

    return sc_segsum


def _tc_body(acc_ref, cnt_ref, atom_ref, wrbf_ref, wpair_ref, brbf_ref,
             bpair_ref, wa1_ref, ba1_ref, wa2_ref, ba2_ref, out_ref):
    f32 = jnp.float32
    wc = jnp.dot(wrbf_ref[...], wpair_ref[...], preferred_element_type=f32)
    w1 = jnp.dot(wc, wa1_ref[...], preferred_element_type=f32)
    bc = jnp.dot(brbf_ref[...], wpair_ref[...], preferred_element_type=f32) + bpair_ref[...]
    b1 = jnp.dot(bc, wa1_ref[...], preferred_element_type=f32)
    seg = acc_ref[0] + acc_ref[1]
    cnt = cnt_ref[0][:, 0:1] + cnt_ref[1][:, 0:1]
    h = jnp.dot(seg, w1, preferred_element_type=f32) + cnt * b1 + ba1_ref[...]
    h = h * jax.nn.sigmoid(h)
    out_ref[...] = (atom_ref[...]
                    + jnp.dot(h, wa2_ref[...], preferred_element_type=f32)
                    + ba2_ref[...])


@functools.lru_cache(maxsize=None)
def _make_tc_mlp(n_nodes: int, d_edge: int, hidden: int, blk: int):
    grid = n_nodes // blk
    assert grid * blk == n_nodes
    full = lambda shape: pl.BlockSpec(shape, lambda i: (0,) * len(shape))
    return pl.pallas_call(
        _tc_body,
        grid=(grid,),
        in_specs=[
            pl.BlockSpec((2, blk, d_edge), lambda i: (0, i, 0)),
            pl.BlockSpec((2, blk, d_edge), lambda i: (0, i, 0)),
            pl.BlockSpec((blk, hidden), lambda i: (i, 0)),
            full((d_edge, hidden)),
            full((hidden, hidden)),
            full((1, hidden)),
            full((1, hidden)),
            full((hidden, hidden)),
            full((1, hidden)),
            full((hidden, hidden)),
            full((1, hidden)),
        ],
        out_specs=pl.BlockSpec((blk, hidden), lambda i: (i, 0)),
        out_shape=jax.ShapeDtypeStruct((n_nodes, hidden), jnp.float32),
    )


def kernel(atom_feat, pair_feat, recv_idx, W_rbf, b_rbf, W_pair, b_pair,
           W_a1, b_a1, W_a2, b_a2):
    n_nodes, hidden = atom_feat.shape
    n_edges, d_edge = pair_feat.shape
    idx = recv_idx.astype(jnp.int32)
    half = (n_edges // 2) // _SCATTER_BATCH * _SCATTER_BATCH
    acc, cnt = _make_sc_segsum(half, n_nodes, d_edge, False)(
        pair_feat[:half], idx[:half])
    acc, cnt = _make_sc_segsum(n_edges - half, n_nodes, d_edge, True)(
        pair_feat[half:], idx[half:], acc, cnt)
    tc = _make_tc_mlp(n_nodes, d_edge, hidden, 1000)
    return tc(acc, cnt, atom_feat, W_rbf, W_pair,
              b_rbf.reshape(1, hidden), b_pair.reshape(1, hidden),
              W_a1, b_a1.reshape(1, hidden), W_a2, b_a2.reshape(1, hidden))


# speedup vs baseline: 1.1123x; 1.1123x over previous
import functools

import jax
import jax.numpy as jnp
from jax import lax
from jax.experimental import pallas as pl
from jax.experimental.pallas import tpu as pltpu
from jax.experimental.pallas import tpu_sc as plsc

_LANES = 16
_SCATTER_BATCH = 128


@functools.lru_cache(maxsize=None)
def _make_sc_segsum(n_edges: int, n_nodes: int, d_edge: int):
    info = plsc.get_sparse_core_info()
    nc, ns = info.num_cores, info.num_subcores
    nw = nc * ns
    wide = 2 * d_edge
    rows = n_edges // _SCATTER_BATCH
    assert rows * _SCATTER_BATCH == n_edges
    rpt = rows // nw
    extra = rows - rpt * nw
    stripe = (n_nodes // ns) // 8 * 8
    tail = n_nodes - ns * stripe
    assert tail % 8 == 0
    hs = 8
    chunks = []
    r = 0
    while r < rpt:
        n = min(hs, rpt - r)
        chunks.append((r, n))
        r += n

    mesh = plsc.VectorSubcoreMesh(core_axis_name="c", subcore_axis_name="s")
    f32 = jnp.float32

    @functools.partial(
        pl.kernel,
        mesh=mesh,
        compiler_params=pltpu.CompilerParams(use_tc_tiling_on_sc=False),
        out_type=jax.ShapeDtypeStruct((nc, n_nodes, wide), f32),
        scratch_types=[
            pltpu.VMEM_SHARED((n_nodes, wide), f32),
            pltpu.VMEM((rpt * _SCATTER_BATCH,), jnp.int32),
            pltpu.VMEM((2, hs * _SCATTER_BATCH, wide), f32),
            pltpu.VMEM((stripe, wide), f32),
            pltpu.VMEM((_SCATTER_BATCH,), jnp.int32),
            pltpu.VMEM((_SCATTER_BATCH, wide), f32),
            pltpu.SemaphoreType.DMA,
            pltpu.SemaphoreType.DMA,
        ],
    )
    def sc_segsum(pair_hbm, idx_hbm, acc_out,
                  acc_sh, idx_v, pair_v, zero_v, eidx_v, epair_v, lsem, ssem):
        c = lax.axis_index("c")
        s = lax.axis_index("s")
        wid = c * ns + s

        idx_h = pltpu.async_copy(
            idx_hbm.at[pl.ds(wid * rpt * _SCATTER_BATCH, rpt * _SCATTER_BATCH)],
            idx_v, lsem)

        def start_load(ci):
            r0, n = chunks[ci]
            return pltpu.async_copy(
                pair_hbm.at[pl.ds((wid * rpt + r0) * _SCATTER_BATCH,
                                  n * _SCATTER_BATCH)],
                pair_v.at[ci % 2, pl.ds(0, n * _SCATTER_BATCH),
                          pl.ds(0, d_edge)], lsem)

        loads = [start_load(0), None]

        zeros16 = jnp.zeros((_LANES,), f32)
        ones16 = jnp.ones((_LANES,), f32)

        def fillo(i, _):
            pair_v[0, i, pl.ds(d_edge, _LANES)] = ones16
            pair_v[1, i, pl.ds(d_edge, _LANES)] = ones16
            return 0
        lax.fori_loop(0, hs * _SCATTER_BATCH, fillo, 0)

        def fillz(i, _):
            zero_v[i, pl.ds(0, _LANES)] = zeros16
            zero_v[i, pl.ds(_LANES, _LANES)] = zeros16
            return 0
        lax.fori_loop(0, stripe, fillz, 0)

        @pl.when(wid < extra)
        def _():
            def fille(i, _):
                epair_v[i, pl.ds(d_edge, _LANES)] = ones16
                return 0
            lax.fori_loop(0, _SCATTER_BATCH, fille, 0)

        pltpu.sync_copy(zero_v, acc_sh.at[pl.ds(s * stripe, stripe)])

        @pl.when(s == 0)
        def _():
            pltpu.sync_copy(zero_v.at[pl.ds(0, tail)],
                            acc_sh.at[pl.ds(ns * stripe, tail)])

        plsc.subcore_barrier()
        idx_h.wait()

        for ci, (r0, n) in enumerate(chunks):
            loads[ci % 2].wait()
            if ci + 1 < len(chunks):
                loads[(ci + 1) % 2] = start_load(ci + 1)
            handles = []
            for j in range(n):
                idx_row = idx_v.at[pl.ds((r0 + j) * _SCATTER_BATCH,
                                         _SCATTER_BATCH)]
                handles.append(pltpu.async_copy(
                    pair_v.at[ci % 2, pl.ds(j * _SCATTER_BATCH, _SCATTER_BATCH)],
                    acc_sh.at[idx_row], ssem, add=True))
            for h in handles:
                h.wait()

        if extra:
            @pl.when(wid < extra)
            def _():
                e_base = (nw * rpt + wid) * _SCATTER_BATCH
                pltpu.sync_copy(idx_hbm.at[pl.ds(e_base, _SCATTER_BATCH)], eidx_v)
                pltpu.sync_copy(pair_hbm.at[pl.ds(e_base, _SCATTER_BATCH)],
                                epair_v.at[:, pl.ds(0, d_edge)])
                pltpu.sync_copy(epair_v, acc_sh.at[eidx_v], add=True)

        plsc.subcore_barrier()

        pltpu.sync_copy(acc_sh.at[pl.ds(s * stripe, stripe)],
                        acc_out.at[c, pl.ds(s * stripe, stripe)])

        @pl.when(s == 0)
        def _():
            pltpu.sync_copy(acc_sh.at[pl.ds(ns * stripe, tail)],
                            acc_out.at[c, pl.ds(ns * stripe, tail)])

    return sc_segsum


def _tc_body(acc_ref, atom_ref, wrbf_ref, wpair_ref, brbf_ref,
             bpair_ref, wa1_ref, ba1_ref, wa2_ref, ba2_ref, out_ref):
    f32 = jnp.float32
    d_edge = wrbf_ref.shape[0]
    wc = jnp.dot(wrbf_ref[...], wpair_ref[...], preferred_element_type=f32)
    w1 = jnp.dot(wc, wa1_ref[...], preferred_element_type=f32)
    bc = jnp.dot(brbf_ref[...], wpair_ref[...], preferred_element_type=f32) + bpair_ref[...]
    b1 = jnp.dot(bc, wa1_ref[...], preferred_element_type=f32)
    both = acc_ref[0] + acc_ref[1]
    seg = both[:, 0:d_edge]
    cnt = both[:, d_edge:d_edge + 1]
    h = jnp.dot(seg, w1, preferred_element_type=f32) + cnt * b1 + ba1_ref[...]
    h = h * jax.nn.sigmoid(h)
    out_ref[...] = (atom_ref[...]
                    + jnp.dot(h, wa2_ref[...], preferred_element_type=f32)
                    + ba2_ref[...])


@functools.lru_cache(maxsize=None)
def _make_tc_mlp(n_nodes: int, d_edge: int, hidden: int, blk: int):
    grid = n_nodes // blk
    assert grid * blk == n_nodes
    full = lambda shape: pl.BlockSpec(shape, lambda i: (0,) * len(shape))
    return pl.pallas_call(
        _tc_body,
        grid=(grid,),
        in_specs=[
            pl.BlockSpec((2, blk, 2 * d_edge), lambda i: (0, i, 0)),
            pl.BlockSpec((blk, hidden), lambda i: (i, 0)),
            full((d_edge, hidden)),
            full((hidden, hidden)),
            full((1, hidden)),
            full((1, hidden)),
            full((hidden, hidden)),
            full((1, hidden)),
            full((hidden, hidden)),
            full((1, hidden)),
        ],
        out_specs=pl.BlockSpec((blk, hidden), lambda i: (i, 0)),
        out_shape=jax.ShapeDtypeStruct((n_nodes, hidden), jnp.float32),
    )


def kernel(atom_feat, pair_feat, recv_idx, W_rbf, b_rbf, W_pair, b_pair,
           W_a1, b_a1, W_a2, b_a2):
    n_nodes, hidden = atom_feat.shape
    n_edges, d_edge = pair_feat.shape
    idx = recv_idx.astype(jnp.int32)
    acc = _make_sc_segsum(n_edges, n_nodes, d_edge)(pair_feat, idx)
    tc = _make_tc_mlp(n_nodes, d_edge, hidden, 1000)
    return tc(acc, atom_feat, W_rbf, W_pair,
              b_rbf.reshape(1, hidden), b_pair.reshape(1, hidden),
              W_a1, b_a1.reshape(1, hidden), W_a2, b_a2.reshape(1, hidden))


# speedup vs baseline: 1.3482x; 1.2121x over previous
import functools

import jax
import jax.numpy as jnp
from jax import lax
from jax.experimental import pallas as pl
from jax.experimental.pallas import tpu as pltpu
from jax.experimental.pallas import tpu_sc as plsc

_LANES = 16
_SCATTER_BATCH = 128


@functools.lru_cache(maxsize=None)
def _make_sc_segsum(n_edges: int, n_nodes: int, d_edge: int):
    info = plsc.get_sparse_core_info()
    nc, ns = info.num_cores, info.num_subcores
    nw = nc * ns
    rows = n_edges // _SCATTER_BATCH
    assert rows * _SCATTER_BATCH == n_edges
    rpt = rows // nw
    extra = rows - rpt * nw
    stripe = (n_nodes // ns) // 8 * 8
    tail = n_nodes - ns * stripe
    assert tail % 8 == 0
    hs = 13
    chunks = []
    r = 0
    while r < rpt:
        n = min(hs, rpt - r)
        chunks.append((r, n))
        r += n

    mesh = plsc.VectorSubcoreMesh(core_axis_name="c", subcore_axis_name="s")
    f32 = jnp.float32

    @functools.partial(
        pl.kernel,
        mesh=mesh,
        compiler_params=pltpu.CompilerParams(use_tc_tiling_on_sc=False),
        out_type=(
            jax.ShapeDtypeStruct((nc, n_nodes, d_edge), f32),
            jax.ShapeDtypeStruct((nc, n_nodes, d_edge), f32),
        ),
        scratch_types=[
            pltpu.VMEM_SHARED((n_nodes, d_edge), f32),
            pltpu.VMEM_SHARED((n_nodes, d_edge), f32),
            pltpu.VMEM((rpt * _SCATTER_BATCH,), jnp.int32),
            pltpu.VMEM((2, hs * _SCATTER_BATCH, d_edge), f32),
            pltpu.VMEM((_SCATTER_BATCH, d_edge), f32),
            pltpu.VMEM((stripe, d_edge), f32),
            pltpu.VMEM((_SCATTER_BATCH,), jnp.int32),
            pltpu.VMEM((_SCATTER_BATCH, d_edge), f32),
            pltpu.SemaphoreType.DMA,
            pltpu.SemaphoreType.DMA,
        ],
    )
    def sc_segsum(pair_hbm, idx_hbm, acc_out, cnt_out,
                  acc_sh, cnt_sh, idx_v, pair_v, ones_v, zero_v, eidx_v,
                  epair_v, lsem, ssem):
        c = lax.axis_index("c")
        s = lax.axis_index("s")
        wid = c * ns + s

        idx_h = pltpu.async_copy(
            idx_hbm.at[pl.ds(wid * rpt * _SCATTER_BATCH, rpt * _SCATTER_BATCH)],
            idx_v, lsem)

        def start_load(ci):
            r0, n = chunks[ci]
            return pltpu.async_copy(
                pair_hbm.at[pl.ds((wid * rpt + r0) * _SCATTER_BATCH,
                                  n * _SCATTER_BATCH)],
                pair_v.at[ci % 2, pl.ds(0, n * _SCATTER_BATCH)], lsem)

        loads = [start_load(0), None]

        def fill(i, _):
            zero_v[i] = jnp.zeros((_LANES,), f32)
            return 0
        lax.fori_loop(0, stripe, fill, 0)

        def fill1(i, _):
            ones_v[i] = jnp.ones((_LANES,), f32)
            return 0
        lax.fori_loop(0, _SCATTER_BATCH, fill1, 0)

        pltpu.sync_copy(zero_v, acc_sh.at[pl.ds(s * stripe, stripe)])
        pltpu.sync_copy(zero_v, cnt_sh.at[pl.ds(s * stripe, stripe)])

        @pl.when(s == 0)
        def _():
            pltpu.sync_copy(zero_v.at[pl.ds(0, tail)],
                            acc_sh.at[pl.ds(ns * stripe, tail)])
            pltpu.sync_copy(zero_v.at[pl.ds(0, tail)],
                            cnt_sh.at[pl.ds(ns * stripe, tail)])

        plsc.subcore_barrier()
        idx_h.wait()

        pending = [[], []]
        for ci, (r0, n) in enumerate(chunks):
            loads[ci % 2].wait()
            if ci + 1 < len(chunks):
                for h in pending[(ci + 1) % 2]:
                    h.wait()
                pending[(ci + 1) % 2] = []
                loads[(ci + 1) % 2] = start_load(ci + 1)
            handles = []
            for j in range(n):
                idx_row = idx_v.at[pl.ds((r0 + j) * _SCATTER_BATCH,
                                         _SCATTER_BATCH)]
                handles.append(pltpu.async_copy(
                    pair_v.at[ci % 2, pl.ds(j * _SCATTER_BATCH, _SCATTER_BATCH)],
                    acc_sh.at[idx_row], ssem, add=True))
                handles.append(pltpu.async_copy(
                    ones_v, cnt_sh.at[idx_row], ssem, add=True))
            pending[ci % 2] = handles
        for h in pending[0] + pending[1]:
            h.wait()

        if extra:
            @pl.when(wid < extra)
            def _():
                e_base = (nw * rpt + wid) * _SCATTER_BATCH
                pltpu.sync_copy(idx_hbm.at[pl.ds(e_base, _SCATTER_BATCH)], eidx_v)
                pltpu.sync_copy(pair_hbm.at[pl.ds(e_base, _SCATTER_BATCH)],
                                epair_v)
                pltpu.sync_copy(epair_v, acc_sh.at[eidx_v], add=True)
                pltpu.sync_copy(ones_v, cnt_sh.at[eidx_v], add=True)

        plsc.subcore_barrier()

        pltpu.sync_copy(acc_sh.at[pl.ds(s * stripe, stripe)],
                        acc_out.at[c, pl.ds(s * stripe, stripe)])
        pltpu.sync_copy(cnt_sh.at[pl.ds(s * stripe, stripe)],
                        cnt_out.at[c, pl.ds(s * stripe, stripe)])

        @pl.when(s == 0)
        def _():
            pltpu.sync_copy(acc_sh.at[pl.ds(ns * stripe, tail)],
                            acc_out.at[c, pl.ds(ns * stripe, tail)])
            pltpu.sync_copy(cnt_sh.at[pl.ds(ns * stripe, tail)],
                            cnt_out.at[c, pl.ds(ns * stripe, tail)])

    return sc_segsum


def _tc_body(acc_ref, cnt_ref, atom_ref, wrbf_ref, wpair_ref, brbf_ref,
             bpair_ref, wa1_ref, ba1_ref, wa2_ref, ba2_ref, out_ref):
    f32 = jnp.float32
    wc = jnp.dot(wrbf_ref[...], wpair_ref[...], preferred_element_type=f32)
    w1 = jnp.dot(wc, wa1_ref[...], preferred_element_type=f32)
    bc = jnp.dot(brbf_ref[...], wpair_ref[...], preferred_element_type=f32) + bpair_ref[...]
    b1 = jnp.dot(bc, wa1_ref[...], preferred_element_type=f32)
    seg = acc_ref[0] + acc_ref[1]
    cnt = cnt_ref[0][:, 0:1] + cnt_ref[1][:, 0:1]
    h = jnp.dot(seg, w1, preferred_element_type=f32) + cnt * b1 + ba1_ref[...]
    h = h * jax.nn.sigmoid(h)
    out_ref[...] = (atom_ref[...]
                    + jnp.dot(h, wa2_ref[...], preferred_element_type=f32)
                    + ba2_ref[...])


@functools.lru_cache(maxsize=None)
def _make_tc_mlp(n_nodes: int, d_edge: int, hidden: int, blk: int):
    grid = n_nodes // blk
    assert grid * blk == n_nodes
    full = lambda shape: pl.BlockSpec(shape, lambda i: (0,) * len(shape))
    return pl.pallas_call(
        _tc_body,
        grid=(grid,),
        in_specs=[
            pl.BlockSpec((2, blk, d_edge), lambda i: (0, i, 0)),
            pl.BlockSpec((2, blk, d_edge), lambda i: (0, i, 0)),
            pl.BlockSpec((blk, hidden), lambda i: (i, 0)),
            full((d_edge, hidden)),
            full((hidden, hidden)),
            full((1, hidden)),
            full((1, hidden)),
            full((hidden, hidden)),
            full((1, hidden)),
            full((hidden, hidden)),
            full((1, hidden)),
        ],
        out_specs=pl.BlockSpec((blk, hidden), lambda i: (i, 0)),
        out_shape=jax.ShapeDtypeStruct((n_nodes, hidden), jnp.float32),
    )


def kernel(atom_feat, pair_feat, recv_idx, W_rbf, b_rbf, W_pair, b_pair,
           W_a1, b_a1, W_a2, b_a2):
    n_nodes, hidden = atom_feat.shape
    n_edges, d_edge = pair_feat.shape
    idx = recv_idx.astype(jnp.int32)
    acc, cnt = _make_sc_segsum(n_edges, n_nodes, d_edge)(pair_feat, idx)
    tc = _make_tc_mlp(n_nodes, d_edge, hidden, 1000)
    return tc(acc, cnt, atom_feat, W_rbf, W_pair,
              b_rbf.reshape(1, hidden), b_pair.reshape(1, hidden),
              W_a1, b_a1.reshape(1, hidden), W_a2, b_a2.reshape(1, hidden))


# speedup vs baseline: 1.3727x; 1.0182x over previous
import functools

import jax
import jax.numpy as jnp
from jax import lax
from jax.experimental import pallas as pl
from jax.experimental.pallas import tpu as pltpu
from jax.experimental.pallas import tpu_sc as plsc

_LANES = 16
_SCATTER_BATCH = 128


@functools.lru_cache(maxsize=None)
def _make_sc_segsum(n_edges: int, n_nodes: int, d_edge: int):
    info = plsc.get_sparse_core_info()
    nc, ns = info.num_cores, info.num_subcores
    nw = nc * ns
    rows = n_edges // _SCATTER_BATCH
    assert rows * _SCATTER_BATCH == n_edges
    rpt = rows // nw
    extra = rows - rpt * nw
    stripe = (n_nodes // ns) // 8 * 8
    tail = n_nodes - ns * stripe
    assert tail % 8 == 0
    hs = 16
    chunks = []
    r = 0
    while r < rpt:
        n = min(hs, rpt - r)
        chunks.append((r, n))
        r += n

    mesh = plsc.VectorSubcoreMesh(core_axis_name="c", subcore_axis_name="s")
    f32 = jnp.float32

    @functools.partial(
        pl.kernel,
        mesh=mesh,
        compiler_params=pltpu.CompilerParams(use_tc_tiling_on_sc=False),
        out_type=(
            jax.ShapeDtypeStruct((nc, n_nodes, d_edge), f32),
            jax.ShapeDtypeStruct((nc, n_nodes, d_edge), f32),
        ),
        scratch_types=[
            pltpu.VMEM_SHARED((n_nodes, d_edge), f32),
            pltpu.VMEM_SHARED((n_nodes, d_edge), f32),
            pltpu.VMEM((rpt * _SCATTER_BATCH,), jnp.int32),
            pltpu.VMEM((2, hs * _SCATTER_BATCH, d_edge), f32),
            pltpu.VMEM((_SCATTER_BATCH, d_edge), f32),
            pltpu.VMEM((stripe, d_edge), f32),
            pltpu.VMEM((_SCATTER_BATCH,), jnp.int32),
            pltpu.VMEM((_SCATTER_BATCH, d_edge), f32),
            pltpu.SemaphoreType.DMA,
            pltpu.SemaphoreType.DMA,
        ],
    )
    def sc_segsum(pair_hbm, idx_hbm, acc_out, cnt_out,
                  acc_sh, cnt_sh, idx_v, pair_v, ones_v, zero_v, eidx_v,
                  epair_v, lsem, ssem):
        c = lax.axis_index("c")
        s = lax.axis_index("s")
        wid = c * ns + s

        idx_h = pltpu.async_copy(
            idx_hbm.at[pl.ds(wid * rpt * _SCATTER_BATCH, rpt * _SCATTER_BATCH)],
            idx_v, lsem)

        def start_load(ci):
            r0, n = chunks[ci]
            return pltpu.async_copy(
                pair_hbm.at[pl.ds((wid * rpt + r0) * _SCATTER_BATCH,
                                  n * _SCATTER_BATCH)],
                pair_v.at[ci % 2, pl.ds(0, n * _SCATTER_BATCH)], lsem)

        loads = [start_load(0), None]

        def fill(i, _):
            zero_v[i] = jnp.zeros((_LANES,), f32)
            return 0
        lax.fori_loop(0, stripe, fill, 0)

        def fill1(i, _):
            ones_v[i] = jnp.ones((_LANES,), f32)
            return 0
        lax.fori_loop(0, _SCATTER_BATCH, fill1, 0)

        pltpu.sync_copy(zero_v, acc_sh.at[pl.ds(s * stripe, stripe)])
        pltpu.sync_copy(zero_v, cnt_sh.at[pl.ds(s * stripe, stripe)])

        @pl.when(s == 0)
        def _():
            pltpu.sync_copy(zero_v.at[pl.ds(0, tail)],
                            acc_sh.at[pl.ds(ns * stripe, tail)])
            pltpu.sync_copy(zero_v.at[pl.ds(0, tail)],
                            cnt_sh.at[pl.ds(ns * stripe, tail)])

        plsc.subcore_barrier()
        idx_h.wait()

        pending = [[], []]
        for ci, (r0, n) in enumerate(chunks):
            loads[ci % 2].wait()
            if ci + 1 < len(chunks):
                for h in pending[(ci + 1) % 2]:
                    h.wait()
                pending[(ci + 1) % 2] = []
                loads[(ci + 1) % 2] = start_load(ci + 1)
            handles = []
            for j in range(n):
                idx_row = idx_v.at[pl.ds((r0 + j) * _SCATTER_BATCH,
                                         _SCATTER_BATCH)]
                handles.append(pltpu.async_copy(
                    pair_v.at[ci % 2, pl.ds(j * _SCATTER_BATCH, _SCATTER_BATCH)],
                    acc_sh.at[idx_row], ssem, add=True))
                handles.append(pltpu.async_copy(
                    ones_v, cnt_sh.at[idx_row], ssem, add=True))
            pending[ci % 2] = handles
        for h in pending[0] + pending[1]:
            h.wait()

        if extra:
            @pl.when(wid < extra)
            def _():
                e_base = (nw * rpt + wid) * _SCATTER_BATCH
                pltpu.sync_copy(idx_hbm.at[pl.ds(e_base, _SCATTER_BATCH)], eidx_v)
                pltpu.sync_copy(pair_hbm.at[pl.ds(e_base, _SCATTER_BATCH)],
                                epair_v)
                pltpu.sync_copy(epair_v, acc_sh.at[eidx_v], add=True)
                pltpu.sync_copy(ones_v, cnt_sh.at[eidx_v], add=True)

        plsc.subcore_barrier()

        pltpu.sync_copy(acc_sh.at[pl.ds(s * stripe, stripe)],
                        acc_out.at[c, pl.ds(s * stripe, stripe)])
        pltpu.sync_copy(cnt_sh.at[pl.ds(s * stripe, stripe)],
                        cnt_out.at[c, pl.ds(s * stripe, stripe)])

        @pl.when(s == 0)
        def _():
            pltpu.sync_copy(acc_sh.at[pl.ds(ns * stripe, tail)],
                            acc_out.at[c, pl.ds(ns * stripe, tail)])
            pltpu.sync_copy(cnt_sh.at[pl.ds(ns * stripe, tail)],
                            cnt_out.at[c, pl.ds(ns * stripe, tail)])

    return sc_segsum


def _tc_body(acc_ref, cnt_ref, atom_ref, wrbf_ref, wpair_ref, brbf_ref,
             bpair_ref, wa1_ref, ba1_ref, wa2_ref, ba2_ref, out_ref):
    f32 = jnp.float32
    wc = jnp.dot(wrbf_ref[...], wpair_ref[...], preferred_element_type=f32)
    w1 = jnp.dot(wc, wa1_ref[...], preferred_element_type=f32)
    bc = jnp.dot(brbf_ref[...], wpair_ref[...], preferred_element_type=f32) + bpair_ref[...]
    b1 = jnp.dot(bc, wa1_ref[...], preferred_element_type=f32)
    seg = acc_ref[0] + acc_ref[1]
    cnt = cnt_ref[0][:, 0:1] + cnt_ref[1][:, 0:1]
    h = jnp.dot(seg, w1, preferred_element_type=f32) + cnt * b1 + ba1_ref[...]
    h = h * jax.nn.sigmoid(h)
    out_ref[...] = (atom_ref[...]
                    + jnp.dot(h, wa2_ref[...], preferred_element_type=f32)
                    + ba2_ref[...])


@functools.lru_cache(maxsize=None)
def _make_tc_mlp(n_nodes: int, d_edge: int, hidden: int, blk: int):
    grid = n_nodes // blk
    assert grid * blk == n_nodes
    full = lambda shape: pl.BlockSpec(shape, lambda i: (0,) * len(shape))
    return pl.pallas_call(
        _tc_body,
        grid=(grid,),
        in_specs=[
            pl.BlockSpec((2, blk, d_edge), lambda i: (0, i, 0)),
            pl.BlockSpec((2, blk, d_edge), lambda i: (0, i, 0)),
            pl.BlockSpec((blk, hidden), lambda i: (i, 0)),
            full((d_edge, hidden)),
            full((hidden, hidden)),
            full((1, hidden)),
            full((1, hidden)),
            full((hidden, hidden)),
            full((1, hidden)),
            full((hidden, hidden)),
            full((1, hidden)),
        ],
        out_specs=pl.BlockSpec((blk, hidden), lambda i: (i, 0)),
        out_shape=jax.ShapeDtypeStruct((n_nodes, hidden), jnp.float32),
    )


def kernel(atom_feat, pair_feat, recv_idx, W_rbf, b_rbf, W_pair, b_pair,
           W_a1, b_a1, W_a2, b_a2):
    n_nodes, hidden = atom_feat.shape
    n_edges, d_edge = pair_feat.shape
    idx = recv_idx.astype(jnp.int32)
    acc, cnt = _make_sc_segsum(n_edges, n_nodes, d_edge)(pair_feat, idx)
    tc = _make_tc_mlp(n_nodes, d_edge, hidden, 2000)
    return tc(acc, cnt, atom_feat, W_rbf, W_pair,
              b_rbf.reshape(1, hidden), b_pair.reshape(1, hidden),
              W_a1, b_a1.reshape(1, hidden), W_a2, b_a2.reshape(1, hidden))


# speedup vs baseline: 1.3761x; 1.0025x over previous
import functools

import jax
import jax.numpy as jnp
from jax import lax
from jax.experimental import pallas as pl
from jax.experimental.pallas import tpu as pltpu
from jax.experimental.pallas import tpu_sc as plsc

_LANES = 16
_SCATTER_BATCH = 128


@functools.lru_cache(maxsize=None)
def _make_sc_segsum(n_edges: int, n_nodes: int, d_edge: int):
    info = plsc.get_sparse_core_info()
    nc, ns = info.num_cores, info.num_subcores
    nw = nc * ns
    rows = n_edges // _SCATTER_BATCH
    assert rows * _SCATTER_BATCH == n_edges
    rpt = rows // nw
    extra = rows - rpt * nw
    stripe = (n_nodes // ns) // 8 * 8
    tail = n_nodes - ns * stripe
    assert tail % 8 == 0
    hs = 20
    chunks = []
    r = 0
    while r < rpt:
        n = min(hs, rpt - r)
        chunks.append((r, n))
        r += n

    mesh = plsc.VectorSubcoreMesh(core_axis_name="c", subcore_axis_name="s")
    f32 = jnp.float32

    @functools.partial(
        pl.kernel,
        mesh=mesh,
        compiler_params=pltpu.CompilerParams(use_tc_tiling_on_sc=False),
        out_type=(
            jax.ShapeDtypeStruct((nc, n_nodes, d_edge), f32),
            jax.ShapeDtypeStruct((nc, n_nodes, d_edge), f32),
        ),
        scratch_types=[
            pltpu.VMEM_SHARED((n_nodes, d_edge), f32),
            pltpu.VMEM_SHARED((n_nodes, d_edge), f32),
            pltpu.VMEM((rpt * _SCATTER_BATCH,), jnp.int32),
            pltpu.VMEM((2, hs * _SCATTER_BATCH, d_edge), f32),
            pltpu.VMEM((_SCATTER_BATCH, d_edge), f32),
            pltpu.VMEM((stripe, d_edge), f32),
            pltpu.VMEM((_SCATTER_BATCH,), jnp.int32),
            pltpu.VMEM((_SCATTER_BATCH, d_edge), f32),
            pltpu.SemaphoreType.DMA,
            pltpu.SemaphoreType.DMA,
        ],
    )
    def sc_segsum(pair_hbm, idx_hbm, acc_out, cnt_out,
                  acc_sh, cnt_sh, idx_v, pair_v, ones_v, zero_v, eidx_v,
                  epair_v, lsem, ssem):
        c = lax.axis_index("c")
        s = lax.axis_index("s")
        wid = c * ns + s

        idx_h = pltpu.async_copy(
            idx_hbm.at[pl.ds(wid * rpt * _SCATTER_BATCH, rpt * _SCATTER_BATCH)],
            idx_v, lsem)

        def start_load(ci):
            r0, n = chunks[ci]
            return pltpu.async_copy(
                pair_hbm.at[pl.ds((wid * rpt + r0) * _SCATTER_BATCH,
                                  n * _SCATTER_BATCH)],
                pair_v.at[ci % 2, pl.ds(0, n * _SCATTER_BATCH)], lsem)

        loads = [start_load(0), None]

        def fill(i, _):
            zero_v[i] = jnp.zeros((_LANES,), f32)
            return 0
        lax.fori_loop(0, stripe, fill, 0)

        def fill1(i, _):
            ones_v[i] = jnp.ones((_LANES,), f32)
            return 0
        lax.fori_loop(0, _SCATTER_BATCH, fill1, 0)

        pltpu.sync_copy(zero_v, acc_sh.at[pl.ds(s * stripe, stripe)])
        pltpu.sync_copy(zero_v, cnt_sh.at[pl.ds(s * stripe, stripe)])

        @pl.when(s == 0)
        def _():
            pltpu.sync_copy(zero_v.at[pl.ds(0, tail)],
                            acc_sh.at[pl.ds(ns * stripe, tail)])
            pltpu.sync_copy(zero_v.at[pl.ds(0, tail)],
                            cnt_sh.at[pl.ds(ns * stripe, tail)])

        plsc.subcore_barrier()
        idx_h.wait()

        pending = [[], []]
        for ci, (r0, n) in enumerate(chunks):
            loads[ci % 2].wait()
            if ci + 1 < len(chunks):
                for h in pending[(ci + 1) % 2]:
                    h.wait()
                pending[(ci + 1) % 2] = []
                loads[(ci + 1) % 2] = start_load(ci + 1)
            handles = []
            for j in range(n):
                idx_row = idx_v.at[pl.ds((r0 + j) * _SCATTER_BATCH,
                                         _SCATTER_BATCH)]
                handles.append(pltpu.async_copy(
                    pair_v.at[ci % 2, pl.ds(j * _SCATTER_BATCH, _SCATTER_BATCH)],
                    acc_sh.at[idx_row], ssem, add=True))
                handles.append(pltpu.async_copy(
                    ones_v, cnt_sh.at[idx_row], ssem, add=True))
            pending[ci % 2] = handles
        for h in pending[0] + pending[1]:
            h.wait()

        if extra:
            @pl.when(wid < extra)
            def _():
                e_base = (nw * rpt + wid) * _SCATTER_BATCH
                pltpu.sync_copy(idx_hbm.at[pl.ds(e_base, _SCATTER_BATCH)], eidx_v)
                pltpu.sync_copy(pair_hbm.at[pl.ds(e_base, _SCATTER_BATCH)],
                                epair_v)
                pltpu.sync_copy(epair_v, acc_sh.at[eidx_v], add=True)
                pltpu.sync_copy(ones_v, cnt_sh.at[eidx_v], add=True)

        plsc.subcore_barrier()

        pltpu.sync_copy(acc_sh.at[pl.ds(s * stripe, stripe)],
                        acc_out.at[c, pl.ds(s * stripe, stripe)])
        pltpu.sync_copy(cnt_sh.at[pl.ds(s * stripe, stripe)],
                        cnt_out.at[c, pl.ds(s * stripe, stripe)])

        @pl.when(s == 0)
        def _():
            pltpu.sync_copy(acc_sh.at[pl.ds(ns * stripe, tail)],
                            acc_out.at[c, pl.ds(ns * stripe, tail)])
            pltpu.sync_copy(cnt_sh.at[pl.ds(ns * stripe, tail)],
                            cnt_out.at[c, pl.ds(ns * stripe, tail)])

    return sc_segsum


def _tc_body(acc_ref, cnt_ref, atom_ref, wrbf_ref, wpair_ref, brbf_ref,
             bpair_ref, wa1_ref, ba1_ref, wa2_ref, ba2_ref, out_ref):
    f32 = jnp.float32
    wc = jnp.dot(wrbf_ref[...], wpair_ref[...], preferred_element_type=f32)
    w1 = jnp.dot(wc, wa1_ref[...], preferred_element_type=f32)
    bc = jnp.dot(brbf_ref[...], wpair_ref[...], preferred_element_type=f32) + bpair_ref[...]
    b1 = jnp.dot(bc, wa1_ref[...], preferred_element_type=f32)
    seg = acc_ref[0] + acc_ref[1]
    cnt = cnt_ref[0][:, 0:1] + cnt_ref[1][:, 0:1]
    h = jnp.dot(seg, w1, preferred_element_type=f32) + cnt * b1 + ba1_ref[...]
    h = h * jax.nn.sigmoid(h)
    out_ref[...] = (atom_ref[...]
                    + jnp.dot(h, wa2_ref[...], preferred_element_type=f32)
                    + ba2_ref[...])


@functools.lru_cache(maxsize=None)
def _make_tc_mlp(n_nodes: int, d_edge: int, hidden: int, blk: int):
    grid = n_nodes // blk
    assert grid * blk == n_nodes
    full = lambda shape: pl.BlockSpec(shape, lambda i: (0,) * len(shape))
    return pl.pallas_call(
        _tc_body,
        grid=(grid,),
        in_specs=[
            pl.BlockSpec((2, blk, d_edge), lambda i: (0, i, 0)),
            pl.BlockSpec((2, blk, d_edge), lambda i: (0, i, 0)),
            pl.BlockSpec((blk, hidden), lambda i: (i, 0)),
            full((d_edge, hidden)),
            full((hidden, hidden)),
            full((1, hidden)),
            full((1, hidden)),
            full((hidden, hidden)),
            full((1, hidden)),
            full((hidden, hidden)),
            full((1, hidden)),
        ],
        out_specs=pl.BlockSpec((blk, hidden), lambda i: (i, 0)),
        out_shape=jax.ShapeDtypeStruct((n_nodes, hidden), jnp.float32),
    )


def kernel(atom_feat, pair_feat, recv_idx, W_rbf, b_rbf, W_pair, b_pair,
           W_a1, b_a1, W_a2, b_a2):
    n_nodes, hidden = atom_feat.shape
    n_edges, d_edge = pair_feat.shape
    idx = recv_idx.astype(jnp.int32)
    acc, cnt = _make_sc_segsum(n_edges, n_nodes, d_edge)(pair_feat, idx)
    tc = _make_tc_mlp(n_nodes, d_edge, hidden, 2000)
    return tc(acc, cnt, atom_feat, W_rbf, W_pair,
              b_rbf.reshape(1, hidden), b_pair.reshape(1, hidden),
              W_a1, b_a1.reshape(1, hidden), W_a2, b_a2.reshape(1, hidden))


# speedup vs baseline: 1.3961x; 1.0145x over previous
import functools

import jax
import jax.numpy as jnp
from jax import lax
from jax.experimental import pallas as pl
from jax.experimental.pallas import tpu as pltpu
from jax.experimental.pallas import tpu_sc as plsc

_LANES = 16
_SCATTER_BATCH = 128


@functools.lru_cache(maxsize=None)
def _make_sc_segsum(n_edges: int, n_nodes: int, d_edge: int):
    info = plsc.get_sparse_core_info()
    nc, ns = info.num_cores, info.num_subcores
    nw = nc * ns
    rows = n_edges // _SCATTER_BATCH
    assert rows * _SCATTER_BATCH == n_edges
    rpt = rows // nw
    extra = rows - rpt * nw
    stripe = (n_nodes // ns) // 8 * 8
    tail = n_nodes - ns * stripe
    assert tail % 8 == 0
    hs = 20
    chunks = []
    r = 0
    while r < rpt:
        n = min(hs, rpt - r)
        chunks.append((r, n))
        r += n

    mesh = plsc.VectorSubcoreMesh(core_axis_name="c", subcore_axis_name="s")
    f32 = jnp.float32

    @functools.partial(
        pl.kernel,
        mesh=mesh,
        compiler_params=pltpu.CompilerParams(use_tc_tiling_on_sc=False),
        out_type=jax.ShapeDtypeStruct((nc, n_nodes, 2 * d_edge), f32),
        scratch_types=[
            pltpu.VMEM_SHARED((n_nodes, d_edge), f32),
            pltpu.VMEM_SHARED((n_nodes, d_edge), f32),
            pltpu.VMEM((rpt * _SCATTER_BATCH,), jnp.int32),
            pltpu.VMEM((2, hs * _SCATTER_BATCH, d_edge), f32),
            pltpu.VMEM((_SCATTER_BATCH, d_edge), f32),
            pltpu.VMEM((stripe, d_edge), f32),
            pltpu.VMEM((_SCATTER_BATCH,), jnp.int32),
            pltpu.VMEM((_SCATTER_BATCH, d_edge), f32),
            pltpu.SemaphoreType.DMA,
            pltpu.SemaphoreType.DMA,
        ],
    )
    def sc_segsum(pair_hbm, idx_hbm, acc_out,
                  acc_sh, cnt_sh, idx_v, pair_v, ones_v, zero_v, eidx_v,
                  epair_v, lsem, ssem):
        c = lax.axis_index("c")
        s = lax.axis_index("s")
        wid = c * ns + s

        idx_h = pltpu.async_copy(
            idx_hbm.at[pl.ds(wid * rpt * _SCATTER_BATCH, rpt * _SCATTER_BATCH)],
            idx_v, lsem)

        def start_load(ci):
            r0, n = chunks[ci]
            return pltpu.async_copy(
                pair_hbm.at[pl.ds((wid * rpt + r0) * _SCATTER_BATCH,
                                  n * _SCATTER_BATCH)],
                pair_v.at[ci % 2, pl.ds(0, n * _SCATTER_BATCH)], lsem)

        loads = [start_load(0), None]

        def fill(i, _):
            zero_v[i] = jnp.zeros((_LANES,), f32)
            return 0
        lax.fori_loop(0, stripe, fill, 0)

        def fill1(i, _):
            ones_v[i] = jnp.ones((_LANES,), f32)
            return 0
        lax.fori_loop(0, _SCATTER_BATCH, fill1, 0)

        pltpu.sync_copy(zero_v, acc_sh.at[pl.ds(s * stripe, stripe)])
        pltpu.sync_copy(zero_v, cnt_sh.at[pl.ds(s * stripe, stripe)])

        @pl.when(s == 0)
        def _():
            pltpu.sync_copy(zero_v.at[pl.ds(0, tail)],
                            acc_sh.at[pl.ds(ns * stripe, tail)])
            pltpu.sync_copy(zero_v.at[pl.ds(0, tail)],
                            cnt_sh.at[pl.ds(ns * stripe, tail)])

        plsc.subcore_barrier()
        idx_h.wait()

        pending = [[], []]
        for ci, (r0, n) in enumerate(chunks):
            loads[ci % 2].wait()
            if ci + 1 < len(chunks):
                for h in pending[(ci + 1) % 2]:
                    h.wait()
                pending[(ci + 1) % 2] = []
                loads[(ci + 1) % 2] = start_load(ci + 1)
            handles = []
            for j in range(n):
                idx_row = idx_v.at[pl.ds((r0 + j) * _SCATTER_BATCH,
                                         _SCATTER_BATCH)]
                handles.append(pltpu.async_copy(
                    pair_v.at[ci % 2, pl.ds(j * _SCATTER_BATCH, _SCATTER_BATCH)],
                    acc_sh.at[idx_row], ssem, add=True))
                handles.append(pltpu.async_copy(
                    ones_v, cnt_sh.at[idx_row], ssem, add=True))
            pending[ci % 2] = handles
        for h in pending[0] + pending[1]:
            h.wait()

        if extra:
            @pl.when(wid < extra)
            def _():
                e_base = (nw * rpt + wid) * _SCATTER_BATCH
                pltpu.sync_copy(idx_hbm.at[pl.ds(e_base, _SCATTER_BATCH)], eidx_v)
                pltpu.sync_copy(pair_hbm.at[pl.ds(e_base, _SCATTER_BATCH)],
                                epair_v)
                pltpu.sync_copy(epair_v, acc_sh.at[eidx_v], add=True)
                pltpu.sync_copy(ones_v, cnt_sh.at[eidx_v], add=True)

        plsc.subcore_barrier()

        pltpu.sync_copy(acc_sh.at[pl.ds(s * stripe, stripe)],
                        acc_out.at[c, pl.ds(s * stripe, stripe),
                                   pl.ds(0, d_edge)])
        pltpu.sync_copy(cnt_sh.at[pl.ds(s * stripe, stripe)],
                        acc_out.at[c, pl.ds(s * stripe, stripe),
                                   pl.ds(d_edge, d_edge)])

        @pl.when(s == 0)
        def _():
            pltpu.sync_copy(acc_sh.at[pl.ds(ns * stripe, tail)],
                            acc_out.at[c, pl.ds(ns * stripe, tail),
                                       pl.ds(0, d_edge)])
            pltpu.sync_copy(cnt_sh.at[pl.ds(ns * stripe, tail)],
                            acc_out.at[c, pl.ds(ns * stripe, tail),
                                       pl.ds(d_edge, d_edge)])

    return sc_segsum


def _tc_body(acc_ref, atom_ref, wrbf_ref, wpair_ref, brbf_ref,
             bpair_ref, wa1_ref, ba1_ref, wa2_ref, ba2_ref, out_ref):
    f32 = jnp.float32
    d_edge = wrbf_ref.shape[0]
    wc = jnp.dot(wrbf_ref[...], wpair_ref[...], preferred_element_type=f32)
    w1 = jnp.dot(wc, wa1_ref[...], preferred_element_type=f32)
    bc = jnp.dot(brbf_ref[...], wpair_ref[...], preferred_element_type=f32) + bpair_ref[...]
    b1 = jnp.dot(bc, wa1_ref[...], preferred_element_type=f32)
    both = acc_ref[0] + acc_ref[1]
    seg = both[:, 0:d_edge]
    cnt = both[:, d_edge:d_edge + 1]
    h = jnp.dot(seg, w1, preferred_element_type=f32) + cnt * b1 + ba1_ref[...]
    h = h * jax.nn.sigmoid(h)
    out_ref[...] = (atom_ref[...]
                    + jnp.dot(h, wa2_ref[...], preferred_element_type=f32)
                    + ba2_ref[...])


@functools.lru_cache(maxsize=None)
def _make_tc_mlp(n_nodes: int, d_edge: int, hidden: int, blk: int):
    grid = n_nodes // blk
    assert grid * blk == n_nodes
    full = lambda shape: pl.BlockSpec(shape, lambda i: (0,) * len(shape))
    return pl.pallas_call(
        _tc_body,
        grid=(grid,),
        in_specs=[
            pl.BlockSpec((2, blk, 2 * d_edge), lambda i: (0, i, 0)),
            pl.BlockSpec((blk, hidden), lambda i: (i, 0)),
            full((d_edge, hidden)),
            full((hidden, hidden)),
            full((1, hidden)),
            full((1, hidden)),
            full((hidden, hidden)),
            full((1, hidden)),
            full((hidden, hidden)),
            full((1, hidden)),
        ],
        out_specs=pl.BlockSpec((blk, hidden), lambda i: (i, 0)),
        out_shape=jax.ShapeDtypeStruct((n_nodes, hidden), jnp.float32),
    )


def kernel(atom_feat, pair_feat, recv_idx, W_rbf, b_rbf, W_pair, b_pair,
           W_a1, b_a1, W_a2, b_a2):
    n_nodes, hidden = atom_feat.shape
    n_edges, d_edge = pair_feat.shape
    idx = recv_idx.astype(jnp.int32)
    acc = _make_sc_segsum(n_edges, n_nodes, d_edge)(pair_feat, idx)
    tc = _make_tc_mlp(n_nodes, d_edge, hidden, 2000)
    return tc(acc, atom_feat, W_rbf, W_pair,
              b_rbf.reshape(1, hidden), b_pair.reshape(1, hidden),
              W_a1, b_a1.reshape(1, hidden), W_a2, b_a2.reshape(1, hidden))
